# baseline (speedup 1.0000x reference)
import functools

import jax
import jax.numpy as jnp
from jax import lax
from jax.experimental import pallas as pl
from jax.experimental.pallas import tpu as pltpu
from jax.experimental.pallas import tpu_sc as plsc

_NC = 2
_NS = 16
_NW = _NC * _NS

_W = 4
_NBUF = 2


def _split_widths(n):
    widths = []
    while n > 0:
        w = min(128, n)
        widths.append(w)
        n -= w
    return widths


@functools.cache
def _build(batch, hist, vocab, d, dtype):
    rows_per_w = batch // _NW
    n_steps = rows_per_w // _W
    assert n_steps % _NBUF == 0
    chunk_idx = _W * hist
    widths = _split_widths(chunk_idx)
    offs = [sum(widths[:j]) for j in range(len(widths))]
    assert all(o % 8 == 0 for o in offs)

    mesh = plsc.VectorSubcoreMesh(
        core_axis_name="c", subcore_axis_name="s",
        num_cores=_NC, num_subcores=_NS,
    )

    @functools.partial(
        pl.kernel,
        out_type=jax.ShapeDtypeStruct((batch, hist, d), dtype),
        mesh=mesh,
        scratch_types=[
            pltpu.VMEM((_NBUF, chunk_idx), jnp.int32),
            pltpu.VMEM((_NBUF, chunk_idx, d), dtype),
            [pltpu.SemaphoreType.DMA] * _NBUF,
            [pltpu.SemaphoreType.DMA] * _NBUF,
            [pltpu.SemaphoreType.DMA] * _NBUF,
        ],
        compiler_params=pltpu.CompilerParams(use_tc_tiling_on_sc=False),
    )
    def gather(idx_hbm, table_hbm, out_hbm, idx_v, rows_v, isems, gsems, osems):
        wid = lax.axis_index("s") * _NC + lax.axis_index("c")
        b0 = wid * rows_per_w

        def gather_refs(b, j):
            src = table_hbm.at[idx_v.at[b].at[pl.ds(offs[j], widths[j])]]
            dst = rows_v.at[b].at[pl.ds(offs[j], widths[j])]
            return src, dst

        def fire_gathers(b):
            for j in range(len(widths)):
                src, dst = gather_refs(b, j)
                pltpu.async_copy(src, dst, gsems[b])

        def wait_gathers(b):
            for j in range(len(widths)):
                src, dst = gather_refs(b, j)
                pltpu.make_async_copy(src, dst, gsems[b]).wait()

        def writeback_refs(b, rb, i):
            src = rows_v.at[b].at[pl.ds(i * hist, hist)]
            dst = out_hbm.at[rb + i]
            return src, dst

        def fire_writebacks(b, rb):
            for i in range(_W):
                src, dst = writeback_refs(b, rb, i)
                pltpu.async_copy(src, dst, osems[b])

        def wait_writebacks(b, rb):
            for i in range(_W):
                src, dst = writeback_refs(b, rb, i)
                pltpu.make_async_copy(src, dst, osems[b]).wait()

        for b in range(_NBUF):
            pltpu.async_copy(
                idx_hbm.at[pl.ds((b0 + b * _W) * hist, chunk_idx)],
                idx_v.at[b], isems[b])

        @pl.loop(0, n_steps, step=_NBUF)
        def _step(g0):
            for b in range(_NBUF):
                g = g0 + b
                rb = b0 + g * _W
                p = (b - 1) % _NBUF

                @pl.when(g0 >= _NBUF)
                def _():
                    wait_writebacks(b, rb)

                pltpu.make_async_copy(
                    idx_hbm.at[pl.ds(rb * hist, chunk_idx)], idx_v.at[b],
                    isems[b]).wait()

                fire_gathers(b)

                @pl.when(g >= 1)
                def _():
                    wait_gathers(p)
                    fire_writebacks(p, rb - _W)

                    @pl.when(g - 1 + _NBUF < n_steps)
                    def _():
                        pltpu.async_copy(
                            idx_hbm.at[
                                pl.ds((rb + (_NBUF - 1) * _W) * hist,
                                      chunk_idx)],
                            idx_v.at[p], isems[p])

        last = (n_steps - 1) % _NBUF
        rb_last = b0 + (n_steps - 1) * _W
        wait_gathers(last)
        fire_writebacks(last, rb_last)
        for b in range(_NBUF):
            wait_writebacks(b, b0)

    return gather


def kernel(x, table):
    batch, hist = x.shape
    vocab, d = table.shape
    flat = x.reshape(-1).astype(jnp.int32)
    assert batch % (_NW * _W) == 0 and hist % 8 == 0
    return _build(batch, hist, vocab, d, table.dtype)(flat, table)
